# baseline (speedup 1.0000x reference)
import functools

import jax
import jax.numpy as jnp
from jax.experimental import pallas as pl
from jax.experimental.pallas import tpu as pltpu

_DEP_DIM = 4
_SLOT_NUM = 4096
_HASH_SEED = 1
_BLOCK_B = 256
_NBUF = 4


def _onehot_block(step, batch_size, block_b):
    blocks_per_d = batch_size // block_b
    d = step // blocks_per_d
    ib = step % blocks_per_d
    b = jax.lax.broadcasted_iota(jnp.int32, (block_b, 1), 0) + ib * block_b
    k = b * _DEP_DIM + d
    m = (k % batch_size) * _DEP_DIM + (k // batch_size)
    h = m.astype(jnp.uint32) * jnp.uint32(2654435761) + jnp.uint32(_HASH_SEED)
    h = h ^ (h >> jnp.uint32(16))
    s = (h % jnp.uint32(_SLOT_NUM)).astype(jnp.int32)
    slots = jax.lax.broadcasted_iota(jnp.int32, (block_b, _SLOT_NUM), 1)
    return (slots == s).astype(jnp.float32)


def _copy_for(step, out_ref, buf_ref, sem_ref, batch_size, block_b):
    blocks_per_d = batch_size // block_b
    d = step // blocks_per_d
    ib = step % blocks_per_d
    slot = jax.lax.rem(step, _NBUF)
    return pltpu.make_async_copy(
        buf_ref.at[slot],
        out_ref.at[d, pl.ds(ib * block_b, block_b), :],
        sem_ref.at[slot],
    )


def _pipeline_kernel(out_ref, buf_ref, sem_ref, *, batch_size, block_b):
    num_steps = _DEP_DIM * (batch_size // block_b)
    i = pl.program_id(0)
    slot = jax.lax.rem(i, _NBUF)

    @pl.when(i >= _NBUF)
    def _():
        _copy_for(i - _NBUF, out_ref, buf_ref, sem_ref,
                  batch_size, block_b).wait()

    buf_ref[slot] = _onehot_block(i, batch_size, block_b)
    _copy_for(i, out_ref, buf_ref, sem_ref, batch_size, block_b).start()

    @pl.when(i == num_steps - 1)
    def _():
        for j in range(_NBUF):
            _copy_for(num_steps - _NBUF + j, out_ref, buf_ref, sem_ref,
                      batch_size, block_b).wait()


def kernel(input_tensor):
    batch_size = input_tensor.shape[0]
    num_steps = _DEP_DIM * (batch_size // _BLOCK_B)
    return pl.pallas_call(
        functools.partial(_pipeline_kernel, batch_size=batch_size,
                          block_b=_BLOCK_B),
        grid=(num_steps,),
        out_specs=pl.BlockSpec(memory_space=pltpu.MemorySpace.HBM),
        out_shape=jax.ShapeDtypeStruct((_DEP_DIM, batch_size, _SLOT_NUM),
                                       jnp.float32),
        scratch_shapes=[
            pltpu.VMEM((_NBUF, _BLOCK_B, _SLOT_NUM), jnp.float32),
            pltpu.SemaphoreType.DMA((_NBUF,)),
        ],
    )()
